# baseline (speedup 1.0000x reference)
import jax
import jax.numpy as jnp
from jax.experimental import pallas as pl
from jax.experimental.pallas import tpu as pltpu

ATT_W = 1.0
REP_W = 1.0
BPOS_W = 10.0
BNEG_SIG_W = 3.0
BNEG_BG_W = 6.0
MARGIN_W = 10.0
THR = 0.5
MARGIN = 0.3
K = 64


def _loss_kernel(beta_ref, sid_ref, cp_ref, emb_ref, acc_ref):
    b = pl.program_id(0)

    @pl.when(b == 0)
    def _init():
        for i in range(8):
            acc_ref[i] = 0.0

    beta_b = beta_ref[0]
    sid = sid_ref[0]
    cp = cp_ref[0]
    emb = emb_ref[0]
    N = beta_b.shape[1]

    valid = (sid >= 0.0).astype(jnp.float32)
    cp_valid = cp * valid
    n_valid = jnp.sum(valid)
    n_cpv = jnp.sum(cp_valid)
    processed = jnp.where((n_valid > 0.0) & (n_cpv > 0.0), 1.0, 0.0)

    p = jax.nn.sigmoid(beta_b)
    log1pexp = jnp.log1p(jnp.exp(-jnp.abs(beta_b)))
    relu_b = jnp.maximum(beta_b, 0.0)
    ce1 = relu_b - beta_b + log1pexp
    bce0 = relu_b + log1pexp
    focal1 = 0.75 * (1.0 - p) * (1.0 - p) * ce1

    ids = jax.lax.broadcasted_iota(jnp.int32, (K, 1), 0).astype(jnp.float32)
    onehot = jnp.where(sid == ids, 1.0, 0.0)
    a_cp = onehot * cp
    cnt_cp = jnp.sum(a_cp, axis=1, keepdims=True)
    inst_size = jnp.sum(onehot, axis=1, keepdims=True)
    use = jnp.where(cnt_cp > 0.0, 1.0, 0.0)

    inst_focal = jnp.sum(a_cp * focal1, axis=1, keepdims=True)
    inst_mean = inst_focal / jnp.maximum(cnt_cp, 1.0)
    pos_accum = jnp.sum(use * inst_size * inst_mean)
    total_w = jnp.sum(use * inst_size)
    pos_bce_b = pos_accum / jnp.maximum(total_w, 1.0)

    non_cp = 1.0 - cp
    ncp_cnt = jnp.sum(non_cp)
    neg_bce_b = jnp.sum(bce0 * non_cp) / jnp.maximum(ncp_cnt, 1.0)

    cp_cnt = jnp.sum(cp)
    pos_margin_b = jnp.sum(jnp.maximum(THR + MARGIN - p, 0.0) * cp) / jnp.maximum(cp_cnt, 1.0)
    neg_margin_b = jnp.sum(jnp.maximum(p - (THR - MARGIN), 0.0) * non_cp) / jnp.maximum(ncp_cnt, 1.0)

    bg = jnp.where(sid == -1.0, 1.0, 0.0)
    bg_cnt = jnp.sum(bg)
    bg_bce = jnp.sum(bce0 * bg) / jnp.maximum(bg_cnt, 1.0)

    beta_loss = (BPOS_W * pos_bce_b + BNEG_SIG_W * neg_bce_b + BNEG_BG_W * bg_bce
                 + MARGIN_W * (pos_margin_b + neg_margin_b))

    embsq = emb * emb
    e2_col = jnp.sum(embsq, axis=1, keepdims=True)
    dn_t = (((1,), (1,)), ((), ()))
    s1 = jax.lax.dot_general(onehot, emb, (((1,), (0,)), ((), ())),
                             preferred_element_type=jnp.float32)
    s2 = jax.lax.dot_general(onehot, e2_col, (((1,), (0,)), ((), ())),
                             preferred_element_type=jnp.float32)
    nidx = jax.lax.broadcasted_iota(jnp.int32, (K, N), 1).astype(jnp.float32)
    first = jnp.min(jnp.where(a_cp > 0.0, nidx, float(N)), axis=1, keepdims=True)
    sel = jnp.where(nidx == first, 1.0, 0.0)
    c = jax.lax.dot_general(sel, emb, (((1,), (0,)), ((), ())),
                            preferred_element_type=jnp.float32)
    c2 = jnp.sum(c * c, axis=1, keepdims=True)
    cdots1 = jnp.sum(c * s1, axis=1, keepdims=True)
    att_sum = s2 - 2.0 * cdots1 + inst_size * c2
    att_mean = att_sum / jnp.maximum(inst_size, 1.0)
    attraction = ATT_W * jnp.sum(use * att_mean)

    T = 256
    nt = N // T
    ones_d = jnp.ones((1, emb.shape[1]), jnp.float32)

    def _wslice(start):
        cpt = cp_ref[0, :, pl.ds(start, T)]
        sidt = sid_ref[0, :, pl.ds(start, T)]
        return cpt * jnp.where(sidt >= 0.0, 1.0, 0.0)

    def _body_ti(ti, acc):
        rows = emb_ref[0, pl.ds(ti * T, T), :]
        e2r = jnp.sum(rows * rows, axis=1, keepdims=True)
        wr = _wslice(ti * T)

        def _body_tj(tj, acc2):
            cols = emb_ref[0, pl.ds(tj * T, T), :]
            e2c = jax.lax.dot_general(ones_d, cols * cols, dn_t,
                                      preferred_element_type=jnp.float32)
            g = jax.lax.dot_general(rows, cols, dn_t,
                                    preferred_element_type=jnp.float32)
            d2 = jnp.maximum(e2r + e2c - 2.0 * g, 0.0)
            e = jnp.exp(-d2)
            wc = _wslice(tj * T)
            r = jax.lax.dot_general(wr, e, (((1,), (0,)), ((), ())),
                                    preferred_element_type=jnp.float32)
            s = jnp.sum(r * wc)
            factor = jnp.where(tj == ti, 1.0, 2.0)
            return acc2 + factor * s

        return jax.lax.fori_loop(ti, nt, _body_tj, acc)

    rep_sum = jax.lax.fori_loop(0, nt, _body_ti, 0.0)
    rep_mean = rep_sum / jnp.where(n_cpv > 1.0, n_cpv * n_cpv, 1.0)
    repulsion = jnp.where(n_cpv > 1.0, REP_W * rep_mean, 0.0)

    batch_loss = beta_loss + attraction + repulsion

    acc_ref[0] = acc_ref[0] + processed * batch_loss
    acc_ref[1] = acc_ref[1] + processed
    acc_ref[2] = processed * pos_bce_b + (1.0 - processed) * acc_ref[2]
    acc_ref[3] = processed * neg_bce_b + (1.0 - processed) * acc_ref[3]
    acc_ref[4] = processed * pos_margin_b + (1.0 - processed) * acc_ref[4]
    acc_ref[5] = processed * neg_margin_b + (1.0 - processed) * acc_ref[5]


def kernel(beta, embed, slice_id, is_cp):
    B, N, D = embed.shape
    beta2 = beta[..., 0].astype(jnp.float32).reshape(B, 1, N)
    sidf = slice_id.astype(jnp.float32).reshape(B, 1, N)
    cpf = is_cp.astype(jnp.float32).reshape(B, 1, N)

    acc = pl.pallas_call(
        _loss_kernel,
        grid=(B,),
        in_specs=[
            pl.BlockSpec((1, 1, N), lambda b: (b, 0, 0)),
            pl.BlockSpec((1, 1, N), lambda b: (b, 0, 0)),
            pl.BlockSpec((1, 1, N), lambda b: (b, 0, 0)),
            pl.BlockSpec((1, N, D), lambda b: (b, 0, 0)),
        ],
        out_specs=pl.BlockSpec(memory_space=pltpu.MemorySpace.SMEM),
        out_shape=jax.ShapeDtypeStruct((8,), jnp.float32),
        compiler_params=pltpu.CompilerParams(
            dimension_semantics=("arbitrary",),
        ),
    )(beta2, sidf, cpf, embed)

    total, cnt = acc[0], acc[1]
    final_loss = jnp.where(cnt > 0.0, total / jnp.where(cnt > 0.0, cnt, 1.0), 0.0)
    return (final_loss, acc[2], acc[3], acc[4], acc[5])


# speedup vs baseline: 1.6414x; 1.6414x over previous
import jax
import jax.numpy as jnp
from jax.experimental import pallas as pl
from jax.experimental.pallas import tpu as pltpu

ATT_W = 1.0
REP_W = 1.0
BPOS_W = 10.0
BNEG_SIG_W = 3.0
BNEG_BG_W = 6.0
MARGIN_W = 10.0
THR = 0.5
MARGIN = 0.3
K = 64


def _loss_kernel(beta_ref, sid_ref, cp_ref, emb_ref, acc_ref):
    b = pl.program_id(0)

    @pl.when(b == 0)
    def _init():
        for i in range(8):
            acc_ref[i] = 0.0

    beta_b = beta_ref[0]
    sid = sid_ref[0]
    cp = cp_ref[0]
    emb = emb_ref[0]
    N = beta_b.shape[1]

    valid = (sid >= 0.0).astype(jnp.float32)
    cp_valid = cp * valid
    n_valid = jnp.sum(valid)
    n_cpv = jnp.sum(cp_valid)
    processed = jnp.where((n_valid > 0.0) & (n_cpv > 0.0), 1.0, 0.0)

    p = jax.nn.sigmoid(beta_b)
    log1pexp = jnp.log1p(jnp.exp(-jnp.abs(beta_b)))
    relu_b = jnp.maximum(beta_b, 0.0)
    ce1 = relu_b - beta_b + log1pexp
    bce0 = relu_b + log1pexp
    focal1 = 0.75 * (1.0 - p) * (1.0 - p) * ce1

    ids = jax.lax.broadcasted_iota(jnp.int32, (K, 1), 0).astype(jnp.float32)
    onehot = jnp.where(sid == ids, 1.0, 0.0)
    a_cp = onehot * cp
    cnt_cp = jnp.sum(a_cp, axis=1, keepdims=True)
    inst_size = jnp.sum(onehot, axis=1, keepdims=True)
    use = jnp.where(cnt_cp > 0.0, 1.0, 0.0)

    inst_focal = jnp.sum(a_cp * focal1, axis=1, keepdims=True)
    inst_mean = inst_focal / jnp.maximum(cnt_cp, 1.0)
    pos_accum = jnp.sum(use * inst_size * inst_mean)
    total_w = jnp.sum(use * inst_size)
    pos_bce_b = pos_accum / jnp.maximum(total_w, 1.0)

    non_cp = 1.0 - cp
    ncp_cnt = jnp.sum(non_cp)
    neg_bce_b = jnp.sum(bce0 * non_cp) / jnp.maximum(ncp_cnt, 1.0)

    cp_cnt = jnp.sum(cp)
    pos_margin_b = jnp.sum(jnp.maximum(THR + MARGIN - p, 0.0) * cp) / jnp.maximum(cp_cnt, 1.0)
    neg_margin_b = jnp.sum(jnp.maximum(p - (THR - MARGIN), 0.0) * non_cp) / jnp.maximum(ncp_cnt, 1.0)

    bg = jnp.where(sid == -1.0, 1.0, 0.0)
    bg_cnt = jnp.sum(bg)
    bg_bce = jnp.sum(bce0 * bg) / jnp.maximum(bg_cnt, 1.0)

    beta_loss = (BPOS_W * pos_bce_b + BNEG_SIG_W * neg_bce_b + BNEG_BG_W * bg_bce
                 + MARGIN_W * (pos_margin_b + neg_margin_b))

    embsq = emb * emb
    e2_col = jnp.sum(embsq, axis=1, keepdims=True)
    dn_t = (((1,), (1,)), ((), ()))
    s1 = jax.lax.dot_general(onehot, emb, (((1,), (0,)), ((), ())),
                             preferred_element_type=jnp.float32)
    s2 = jax.lax.dot_general(onehot, e2_col, (((1,), (0,)), ((), ())),
                             preferred_element_type=jnp.float32)
    nidx = jax.lax.broadcasted_iota(jnp.int32, (K, N), 1).astype(jnp.float32)
    first = jnp.min(jnp.where(a_cp > 0.0, nidx, float(N)), axis=1, keepdims=True)
    sel = jnp.where(nidx == first, 1.0, 0.0)
    c = jax.lax.dot_general(sel, emb, (((1,), (0,)), ((), ())),
                            preferred_element_type=jnp.float32)
    c2 = jnp.sum(c * c, axis=1, keepdims=True)
    cdots1 = jnp.sum(c * s1, axis=1, keepdims=True)
    att_sum = s2 - 2.0 * cdots1 + inst_size * c2
    att_mean = att_sum / jnp.maximum(inst_size, 1.0)
    attraction = ATT_W * jnp.sum(use * att_mean)

    T = 256
    nt = N // T
    ones_d = jnp.ones((1, emb.shape[1]), jnp.float32)
    e2_row = jax.lax.dot_general(ones_d, embsq, dn_t,
                                 preferred_element_type=jnp.float32)

    rep_sum = 0.0
    for ti in range(nt):
        rows = emb[ti * T:(ti + 1) * T, :]
        e2r = e2_col[ti * T:(ti + 1) * T, :]
        wr = cp_valid[:, ti * T:(ti + 1) * T]
        for tj in range(ti, nt):
            cols = emb[tj * T:(tj + 1) * T, :]
            e2c = e2_row[:, tj * T:(tj + 1) * T]
            g = jax.lax.dot_general(rows, cols, dn_t,
                                    preferred_element_type=jnp.float32)
            d2 = jnp.maximum(e2r + e2c - 2.0 * g, 0.0)
            e = jnp.exp(-d2)
            wc = cp_valid[:, tj * T:(tj + 1) * T]
            r = jax.lax.dot_general(wr, e, (((1,), (0,)), ((), ())),
                                    preferred_element_type=jnp.float32)
            s = jnp.sum(r * wc)
            rep_sum = rep_sum + (1.0 if ti == tj else 2.0) * s
    rep_mean = rep_sum / jnp.where(n_cpv > 1.0, n_cpv * n_cpv, 1.0)
    repulsion = jnp.where(n_cpv > 1.0, REP_W * rep_mean, 0.0)

    batch_loss = beta_loss + attraction + repulsion

    acc_ref[0] = acc_ref[0] + processed * batch_loss
    acc_ref[1] = acc_ref[1] + processed
    acc_ref[2] = processed * pos_bce_b + (1.0 - processed) * acc_ref[2]
    acc_ref[3] = processed * neg_bce_b + (1.0 - processed) * acc_ref[3]
    acc_ref[4] = processed * pos_margin_b + (1.0 - processed) * acc_ref[4]
    acc_ref[5] = processed * neg_margin_b + (1.0 - processed) * acc_ref[5]


def kernel(beta, embed, slice_id, is_cp):
    B, N, D = embed.shape
    beta2 = beta[..., 0].astype(jnp.float32).reshape(B, 1, N)
    sidf = slice_id.astype(jnp.float32).reshape(B, 1, N)
    cpf = is_cp.astype(jnp.float32).reshape(B, 1, N)

    acc = pl.pallas_call(
        _loss_kernel,
        grid=(B,),
        in_specs=[
            pl.BlockSpec((1, 1, N), lambda b: (b, 0, 0)),
            pl.BlockSpec((1, 1, N), lambda b: (b, 0, 0)),
            pl.BlockSpec((1, 1, N), lambda b: (b, 0, 0)),
            pl.BlockSpec((1, N, D), lambda b: (b, 0, 0)),
        ],
        out_specs=pl.BlockSpec(memory_space=pltpu.MemorySpace.SMEM),
        out_shape=jax.ShapeDtypeStruct((8,), jnp.float32),
        compiler_params=pltpu.CompilerParams(
            dimension_semantics=("arbitrary",),
        ),
    )(beta2, sidf, cpf, embed)

    total, cnt = acc[0], acc[1]
    final_loss = jnp.where(cnt > 0.0, total / jnp.where(cnt > 0.0, cnt, 1.0), 0.0)
    return (final_loss, acc[2], acc[3], acc[4], acc[5])
